# Initial kernel scaffold; baseline (speedup 1.0000x reference)
import jax
import jax.numpy as jnp
from jax.experimental import pallas as pl


def kernel(x1, adj1_index, adj1_weight, x2, adj2_index, adj2_weight, w1a, b1a, w1b, b1b, w2a, b2a, w2b, b2b, wfc1, bfc1, wfc2, bfc2, wfc3, bfc3):
    raise NotImplementedError("write your pallas kernel here")


# speedup vs baseline: 9.8611x; 9.8611x over previous
import functools

import jax
import jax.numpy as jnp
from jax import lax
from jax.experimental import pallas as pl
from jax.experimental.pallas import tpu as pltpu
from jax.experimental.pallas import tpu_sc as plsc

H = 32
G = 125
NPG1 = 116
NPG1P = 120
NPG2 = 200
C = 800
NW = 32


def _rup(x, m):
    return (x + m - 1) // m * m


def _support_body(x1, w1, x2, w2, s1, s2):
    s1[...] = jnp.dot(x1[...], w1[...], preferred_element_type=jnp.float32)
    s2[...] = jnp.dot(x2[...], w2[...], preferred_element_type=jnp.float32)


def _mid_body(p1, b1a, w1b, p2, b2a, w2b, h1, h2):
    a1 = jax.nn.relu(p1[0] + p1[1] + b1a[...])
    h1[...] = jnp.dot(a1, w1b[...], preferred_element_type=jnp.float32)
    a2 = jax.nn.relu(p2[0] + p2[1] + b2a[...])
    h2[...] = jnp.dot(a2, w2b[...], preferred_element_type=jnp.float32)


def _head_body(q1, b1b, q2, b2b, wfc1, bfc1, wfc2, bfc2, wfc3, bfc3, out):
    n1 = G * NPG1P
    h1 = jax.nn.relu(q1[0, :n1] + q1[1, :n1] + b1b[...])
    r1 = h1.reshape(G, NPG1P, H)
    valid = lax.broadcasted_iota(jnp.int32, (G, NPG1P, H), 1) < NPG1
    r1 = jnp.where(valid, r1, 0.0)
    mean1 = jnp.sum(r1, axis=1) * (1.0 / NPG1)
    max1 = jnp.max(r1, axis=1)

    n2 = G * NPG2
    h2 = jax.nn.relu(q2[0, :n2] + q2[1, :n2] + b2b[...])
    r2 = h2.reshape(G, NPG2, H)
    mean2 = jnp.sum(r2, axis=1) * (1.0 / NPG2)
    max2 = jnp.max(r2, axis=1)

    fc = jnp.concatenate([mean1, max1, mean2, max2], axis=1)
    h = jnp.dot(fc, wfc1[...], preferred_element_type=jnp.float32) + bfc1[...]
    h = jnp.dot(h, wfc2[...], preferred_element_type=jnp.float32) + bfc2[...]
    out[...] = jnp.dot(h, wfc3[...], preferred_element_type=jnp.float32) + bfc3[...]


def _tc_call(body, out_shapes, *args):
    return pl.pallas_call(body, out_shape=out_shapes)(*args)


@functools.cache
def _make_sc_agg(n_acc, e, n_tab):
    mesh = plsc.VectorSubcoreMesh(core_axis_name="c", subcore_axis_name="s")
    rpt = n_acc // 16
    nch = e // C

    def body(tab, src, dst, wgt, out, srcv, dstv, wv, rows, zbuf, acc, sem):
        cid = lax.axis_index("c")
        sid = lax.axis_index("s")
        wid = sid * 2 + cid

        zv = jnp.zeros((16,), jnp.float32)

        def zrow(i, _):
            zbuf[i, pl.ds(0, 16)] = zv
            zbuf[i, pl.ds(16, 16)] = zv
            return 0

        lax.fori_loop(0, rpt, zrow, 0, unroll=8)
        pltpu.sync_copy(zbuf.at[pl.ds(0, rpt), :],
                        acc.at[pl.ds(sid * rpt, rpt), :])
        plsc.subcore_barrier()

        def chunk(i, _):
            base = (i * NW + wid) * C
            pltpu.sync_copy(src.at[pl.ds(base, C)], srcv)
            pltpu.sync_copy(dst.at[pl.ds(base, C)], dstv)
            pltpu.sync_copy(wgt.at[pl.ds(base, C)], wv)
            pltpu.async_copy(tab.at[srcv], rows, sem).wait()

            def scale(g, _):
                wvec = wv[pl.ds(g * 16, 16)]
                for j in range(16):
                    ei = g * 16 + j
                    w = wvec[j]
                    rows[ei, pl.ds(0, 16)] = rows[ei, pl.ds(0, 16)] * w
                    rows[ei, pl.ds(16, 16)] = rows[ei, pl.ds(16, 16)] * w
                return 0

            lax.fori_loop(0, C // 16, scale, 0)
            pltpu.sync_copy(rows, acc.at[dstv], add=True)
            return 0

        my = (nch + (NW - 1) - wid) >> 5
        lax.fori_loop(0, my, chunk, 0)
        plsc.subcore_barrier()

        pltpu.sync_copy(acc.at[pl.ds(sid * rpt, rpt), :],
                        out.at[cid, pl.ds(sid * rpt, rpt), :])

    return pl.kernel(
        body,
        out_type=jax.ShapeDtypeStruct((2, n_acc, H), jnp.float32),
        mesh=mesh,
        compiler_params=pltpu.CompilerParams(use_tc_tiling_on_sc=False),
        scratch_types=[
            pltpu.VMEM((C,), jnp.int32),
            pltpu.VMEM((C,), jnp.int32),
            pltpu.VMEM((C,), jnp.float32),
            pltpu.VMEM((C, H), jnp.float32),
            pltpu.VMEM((rpt, H), jnp.float32),
            pltpu.VMEM_SHARED((n_acc, H), jnp.float32),
            pltpu.SemaphoreType.DMA,
        ],
    )


def kernel(x1, adj1_index, adj1_weight, x2, adj2_index, adj2_weight,
           w1a, b1a, w1b, b1b, w2a, b2a, w2b, b2b,
           wfc1, bfc1, wfc2, bfc2, wfc3, bfc3):
    n1, e1 = x1.shape[0], adj1_index.shape[1]
    n2, e2 = x2.shape[0], adj2_index.shape[1]
    n1a = _rup(n1, 128)
    n2a = _rup(n2, 128)
    n1b = _rup(G * NPG1P, 128)

    dst1, src1 = adj1_index[0], adj1_index[1]
    dst2, src2 = adj2_index[0], adj2_index[1]
    dst1p = (dst1 // NPG1) * NPG1P + (dst1 % NPG1)

    s1, s2 = _tc_call(
        _support_body,
        (jax.ShapeDtypeStruct((n1, H), jnp.float32),
         jax.ShapeDtypeStruct((n2, H), jnp.float32)),
        x1, w1a, x2, w2a)

    p1 = _make_sc_agg(n1a, e1, n1)(s1, src1, dst1, adj1_weight)
    p2 = _make_sc_agg(n2a, e2, n2)(s2, src2, dst2, adj2_weight)

    h1, h2 = _tc_call(
        _mid_body,
        (jax.ShapeDtypeStruct((n1a, H), jnp.float32),
         jax.ShapeDtypeStruct((n2a, H), jnp.float32)),
        p1, b1a, w1b, p2, b2a, w2b)

    q1 = _make_sc_agg(n1b, e1, n1a)(h1, src1, dst1p, adj1_weight)
    q2 = _make_sc_agg(n2a, e2, n2a)(h2, src2, dst2, adj2_weight)

    logits = _tc_call(
        _head_body,
        jax.ShapeDtypeStruct((G, 2), jnp.float32),
        q1, b1b, q2, b2b, wfc1, bfc1, wfc2, bfc2, wfc3, bfc3)
    return logits


# speedup vs baseline: 11.7814x; 1.1947x over previous
import functools

import jax
import jax.numpy as jnp
from jax import lax
from jax.experimental import pallas as pl
from jax.experimental.pallas import tpu as pltpu
from jax.experimental.pallas import tpu_sc as plsc

H = 32
G = 125
NPG1 = 116
NPG1P = 120
NPG2 = 200
C = 800
NW = 32


def _rup(x, m):
    return (x + m - 1) // m * m


def _support_body(x1, w1, x2, w2, s1, s2):
    s1[...] = jnp.dot(x1[...], w1[...], preferred_element_type=jnp.float32)
    s2[...] = jnp.dot(x2[...], w2[...], preferred_element_type=jnp.float32)


def _mid_body(p1, b1a, w1b, p2, b2a, w2b, h1, h2):
    a1 = jax.nn.relu(p1[0] + p1[1] + b1a[...])
    h1[...] = jnp.dot(a1, w1b[...], preferred_element_type=jnp.float32)
    a2 = jax.nn.relu(p2[0] + p2[1] + b2a[...])
    h2[...] = jnp.dot(a2, w2b[...], preferred_element_type=jnp.float32)


def _head_body(q1, b1b, q2, b2b, wfc1, bfc1, wfc2, bfc2, wfc3, bfc3, out):
    n1 = G * NPG1P
    h1 = jax.nn.relu(q1[0, :n1] + q1[1, :n1] + b1b[...])
    r1 = h1.reshape(G, NPG1P, H)
    valid = lax.broadcasted_iota(jnp.int32, (G, NPG1P, H), 1) < NPG1
    r1 = jnp.where(valid, r1, 0.0)
    mean1 = jnp.sum(r1, axis=1) * (1.0 / NPG1)
    max1 = jnp.max(r1, axis=1)

    n2 = G * NPG2
    h2 = jax.nn.relu(q2[0, :n2] + q2[1, :n2] + b2b[...])
    r2 = h2.reshape(G, NPG2, H)
    mean2 = jnp.sum(r2, axis=1) * (1.0 / NPG2)
    max2 = jnp.max(r2, axis=1)

    fc = jnp.concatenate([mean1, max1, mean2, max2], axis=1)
    h = jnp.dot(fc, wfc1[...], preferred_element_type=jnp.float32) + bfc1[...]
    h = jnp.dot(h, wfc2[...], preferred_element_type=jnp.float32) + bfc2[...]
    out[...] = jnp.dot(h, wfc3[...], preferred_element_type=jnp.float32) + bfc3[...]


def _tc_call(body, out_shapes, *args):
    return pl.pallas_call(body, out_shape=out_shapes)(*args)


@functools.cache
def _make_sc_agg(n1_acc, n2_acc, e1, e2):
    mesh = plsc.VectorSubcoreMesh(core_axis_name="c", subcore_axis_name="s")
    n_max = max(n1_acc, n2_acc)

    def body(t1, adj1, w1, t2, adj2, w2, zeros, o1, o2,
             idx0, idx1, wv0, wv1, rows0, rows1, acc,
             sg0, sg1, ss0, ss1):
        cid = lax.axis_index("c")
        sid = lax.axis_index("s")
        wid = sid * 2 + cid
        idxb = (idx0, idx1)
        wvb = (wv0, wv1)
        rowsb = (rows0, rows1)
        sg = (sg0, sg1)
        ss = (ss0, ss1)

        def run_branch(tab, adj, wgt, out, n_acc, e):
            rpt = n_acc // 16
            nch = e // C
            k = -(-nch // NW)
            my = (nch + (NW - 1) - wid) >> 5

            pltpu.sync_copy(zeros.at[pl.ds(0, rpt), :],
                            acc.at[pl.ds(sid * rpt, rpt), :])
            plsc.subcore_barrier()

            def fetch_idx(i, b):
                base = (i * NW + wid) * C
                pltpu.sync_copy(adj.at[:, pl.ds(base, C)], idxb[b])
                pltpu.sync_copy(wgt.at[pl.ds(base, C)], wvb[b])

            def gather(b):
                return pltpu.make_async_copy(
                    tab.at[idxb[b].at[1]], rowsb[b], sg[b])

            def scatter(b):
                return pltpu.make_async_copy(
                    rowsb[b], acc.at[idxb[b].at[0]], ss[b])

            def scale(b):
                rows = rowsb[b]
                wv = wvb[b]

                def sgroup(g, _):
                    wvec = wv[pl.ds(g * 16, 16)]
                    for j in range(16):
                        ei = g * 16 + j
                        w = wvec[j]
                        rows[ei, pl.ds(0, 16)] = rows[ei, pl.ds(0, 16)] * w
                        rows[ei, pl.ds(16, 16)] = rows[ei, pl.ds(16, 16)] * w
                    return 0

                lax.fori_loop(0, C // 16, sgroup, 0)

            def step(i, b):
                nb = 1 - b

                @pl.when(jnp.logical_and(i >= 1, i - 1 < my))
                def _():
                    scatter(nb).wait()

                @pl.when(i + 1 < my)
                def _():
                    fetch_idx(i + 1, nb)

                @pl.when(i < my)
                def _():
                    gather(b).wait()

                @pl.when(i + 1 < my)
                def _():
                    gather(nb).start()

                @pl.when(i < my)
                def _():
                    scale(b)
                    scatter(b).start(add=True)

            @pl.when(0 < my)
            def _():
                fetch_idx(0, 0)
                gather(0).start()

            def two_steps(i2, _):
                step(i2 * 2, 0)
                step(i2 * 2 + 1, 1)
                return 0

            lax.fori_loop(0, k // 2, two_steps, 0)
            if k % 2:
                step(k - 1, 0)

            @pl.when(my == k)
            def _():
                scatter((k - 1) % 2).wait()

            plsc.subcore_barrier()
            pltpu.sync_copy(acc.at[pl.ds(sid * rpt, rpt), :],
                            out.at[cid, pl.ds(sid * rpt, rpt), :])
            plsc.subcore_barrier()

        run_branch(t1, adj1, w1, o1, n1_acc, e1)
        run_branch(t2, adj2, w2, o2, n2_acc, e2)

    return pl.kernel(
        body,
        out_type=[
            jax.ShapeDtypeStruct((2, n1_acc, H), jnp.float32),
            jax.ShapeDtypeStruct((2, n2_acc, H), jnp.float32),
        ],
        mesh=mesh,
        compiler_params=pltpu.CompilerParams(use_tc_tiling_on_sc=False),
        scratch_types=[
            pltpu.VMEM((2, C), jnp.int32),
            pltpu.VMEM((2, C), jnp.int32),
            pltpu.VMEM((C,), jnp.float32),
            pltpu.VMEM((C,), jnp.float32),
            pltpu.VMEM((C, H), jnp.float32),
            pltpu.VMEM((C, H), jnp.float32),
            pltpu.VMEM_SHARED((n_max, H), jnp.float32),
            pltpu.SemaphoreType.DMA,
            pltpu.SemaphoreType.DMA,
            pltpu.SemaphoreType.DMA,
            pltpu.SemaphoreType.DMA,
        ],
    )


def kernel(x1, adj1_index, adj1_weight, x2, adj2_index, adj2_weight,
           w1a, b1a, w1b, b1b, w2a, b2a, w2b, b2b,
           wfc1, bfc1, wfc2, bfc2, wfc3, bfc3):
    n1, e1 = x1.shape[0], adj1_index.shape[1]
    n2, e2 = x2.shape[0], adj2_index.shape[1]
    n1a = _rup(n1, 128)
    n2a = _rup(n2, 128)
    n1b = _rup(G * NPG1P, 128)

    dst1p = (adj1_index[0] // NPG1) * NPG1P + (adj1_index[0] % NPG1)
    adj1p = jnp.stack([dst1p, adj1_index[1]])
    zeros = jnp.zeros((max(n1a, n1b, n2a) // 16, H), jnp.float32)

    s1, s2 = _tc_call(
        _support_body,
        (jax.ShapeDtypeStruct((n1, H), jnp.float32),
         jax.ShapeDtypeStruct((n2, H), jnp.float32)),
        x1, w1a, x2, w2a)

    p1, p2 = _make_sc_agg(n1a, n2a, e1, e2)(
        s1, adj1_index, adj1_weight, s2, adj2_index, adj2_weight, zeros)

    h1, h2 = _tc_call(
        _mid_body,
        (jax.ShapeDtypeStruct((n1a, H), jnp.float32),
         jax.ShapeDtypeStruct((n2a, H), jnp.float32)),
        p1, b1a, w1b, p2, b2a, w2b)

    q1, q2 = _make_sc_agg(n1b, n2a, e1, e2)(
        h1, adj1p, adj1_weight, h2, adj2_index, adj2_weight, zeros)

    logits = _tc_call(
        _head_body,
        jax.ShapeDtypeStruct((G, 2), jnp.float32),
        q1, b1b, q2, b2b, wfc1, bfc1, wfc2, bfc2, wfc3, bfc3)
    return logits


# speedup vs baseline: 15.3286x; 1.3011x over previous
import functools

import jax
import jax.numpy as jnp
from jax import lax
from jax.experimental import pallas as pl
from jax.experimental.pallas import tpu as pltpu
from jax.experimental.pallas import tpu_sc as plsc

H = 32
G = 125
NPG1 = 116
NPG1P = 120
NPG2 = 200
C = 800
NW = 32


def _rup(x, m):
    return (x + m - 1) // m * m


def _support_body(x1, w1, x2, w2, s1, s2):
    s1[...] = jnp.dot(x1[...], w1[...], preferred_element_type=jnp.float32)
    s2[...] = jnp.dot(x2[...], w2[...], preferred_element_type=jnp.float32)


def _mid_body(p1, b1a, w1b, p2, b2a, w2b, h1, h2):
    a1 = jax.nn.relu(p1[0] + p1[1] + b1a[...])
    h1[...] = jnp.dot(a1, w1b[...], preferred_element_type=jnp.float32)
    a2 = jax.nn.relu(p2[0] + p2[1] + b2a[...])
    h2[...] = jnp.dot(a2, w2b[...], preferred_element_type=jnp.float32)


def _fold4(a, op):
    return op(op(a[:, 0:32], a[:, 32:64]), op(a[:, 64:96], a[:, 96:128]))


def _head_body(q1, b1b, q2, b2b, wfc1, bfc1, wfc2, bfc2, wfc3, bfc3, out):
    r1 = G * (NPG1P // 4)
    h1 = jax.nn.relu(q1[0, :r1] + q1[1, :r1] + b1b[...])
    h1 = h1.reshape(G, NPG1P // 4, 128)
    valid = lax.broadcasted_iota(jnp.int32, h1.shape, 1) < NPG1 // 4
    h1m = jnp.where(valid, h1, 0.0)
    mean1 = _fold4(jnp.sum(h1m, axis=1), jnp.add) * (1.0 / NPG1)
    max1 = _fold4(jnp.max(h1m, axis=1), jnp.maximum)

    r2 = G * (NPG2 // 4)
    h2 = jax.nn.relu(q2[0, :r2] + q2[1, :r2] + b2b[...])
    h2 = h2.reshape(G, NPG2 // 4, 128)
    mean2 = _fold4(jnp.sum(h2, axis=1), jnp.add) * (1.0 / NPG2)
    max2 = _fold4(jnp.max(h2, axis=1), jnp.maximum)

    fc = jnp.concatenate([mean1, max1, mean2, max2], axis=1)
    h = jnp.dot(fc, wfc1[...], preferred_element_type=jnp.float32) + bfc1[...]
    h = jnp.dot(h, wfc2[...], preferred_element_type=jnp.float32) + bfc2[...]
    out[...] = jnp.dot(h, wfc3[...], preferred_element_type=jnp.float32) + bfc3[...]


def _tc_call(body, out_shapes, *args):
    return pl.pallas_call(body, out_shape=out_shapes)(*args)


@functools.cache
def _make_sc_agg(n1_acc, n2_acc, e1, e2):
    mesh = plsc.VectorSubcoreMesh(core_axis_name="c", subcore_axis_name="s")
    n_max = max(n1_acc, n2_acc)

    def body(t1, adj1, w1, t2, adj2, w2, zeros, o1, o2,
             idx0, idx1, wv0, wv1, rows0, rows1, acc,
             sg0, sg1, ss0, ss1):
        cid = lax.axis_index("c")
        sid = lax.axis_index("s")
        wid = sid * 2 + cid
        idxb = (idx0, idx1)
        wvb = (wv0, wv1)
        rowsb = (rows0, rows1)
        sg = (sg0, sg1)
        ss = (ss0, ss1)

        def run_branch(tab, adj, wgt, out, n_acc, e):
            rpt = n_acc // 16
            nch = e // C
            k = -(-nch // NW)
            my = (nch + (NW - 1) - wid) >> 5

            pltpu.sync_copy(zeros.at[pl.ds(0, rpt), :],
                            acc.at[pl.ds(sid * rpt, rpt), :])
            plsc.subcore_barrier()

            def fetch_idx(i, b):
                base = (i * NW + wid) * C
                pltpu.sync_copy(adj.at[:, pl.ds(base, C)], idxb[b])
                pltpu.sync_copy(wgt.at[pl.ds(base, C)], wvb[b])

            def gather(b):
                return pltpu.make_async_copy(
                    tab.at[idxb[b].at[1]], rowsb[b], sg[b])

            def scatter(b):
                return pltpu.make_async_copy(
                    rowsb[b], acc.at[idxb[b].at[0]], ss[b])

            def scale(b):
                rows = rowsb[b]
                wv = wvb[b]

                def sgroup(g, _):
                    wvec = wv[pl.ds(g * 16, 16)]
                    for j in range(16):
                        ei = g * 16 + j
                        w = wvec[j]
                        rows[ei, pl.ds(0, 16)] = rows[ei, pl.ds(0, 16)] * w
                        rows[ei, pl.ds(16, 16)] = rows[ei, pl.ds(16, 16)] * w
                    return 0

                lax.fori_loop(0, C // 16, sgroup, 0)

            def step(i, b):
                nb = 1 - b

                @pl.when(jnp.logical_and(i >= 1, i - 1 < my))
                def _():
                    scatter(nb).wait()

                @pl.when(i + 1 < my)
                def _():
                    fetch_idx(i + 1, nb)

                @pl.when(i < my)
                def _():
                    gather(b).wait()

                @pl.when(i + 1 < my)
                def _():
                    gather(nb).start()

                @pl.when(i < my)
                def _():
                    scale(b)
                    scatter(b).start(add=True)

            @pl.when(0 < my)
            def _():
                fetch_idx(0, 0)
                gather(0).start()

            def two_steps(i2, _):
                step(i2 * 2, 0)
                step(i2 * 2 + 1, 1)
                return 0

            lax.fori_loop(0, k // 2, two_steps, 0)
            if k % 2:
                step(k - 1, 0)

            @pl.when(my == k)
            def _():
                scatter((k - 1) % 2).wait()

            plsc.subcore_barrier()
            pltpu.sync_copy(acc.at[pl.ds(sid * rpt, rpt), :],
                            out.at[cid, pl.ds(sid * rpt, rpt), :])
            plsc.subcore_barrier()

        run_branch(t1, adj1, w1, o1, n1_acc, e1)
        run_branch(t2, adj2, w2, o2, n2_acc, e2)

    return pl.kernel(
        body,
        out_type=[
            jax.ShapeDtypeStruct((2, n1_acc, H), jnp.float32),
            jax.ShapeDtypeStruct((2, n2_acc, H), jnp.float32),
        ],
        mesh=mesh,
        compiler_params=pltpu.CompilerParams(use_tc_tiling_on_sc=False),
        scratch_types=[
            pltpu.VMEM((2, C), jnp.int32),
            pltpu.VMEM((2, C), jnp.int32),
            pltpu.VMEM((C,), jnp.float32),
            pltpu.VMEM((C,), jnp.float32),
            pltpu.VMEM((C, H), jnp.float32),
            pltpu.VMEM((C, H), jnp.float32),
            pltpu.VMEM_SHARED((n_max, H), jnp.float32),
            pltpu.SemaphoreType.DMA,
            pltpu.SemaphoreType.DMA,
            pltpu.SemaphoreType.DMA,
            pltpu.SemaphoreType.DMA,
        ],
    )


def kernel(x1, adj1_index, adj1_weight, x2, adj2_index, adj2_weight,
           w1a, b1a, w1b, b1b, w2a, b2a, w2b, b2b,
           wfc1, bfc1, wfc2, bfc2, wfc3, bfc3):
    n1, e1 = x1.shape[0], adj1_index.shape[1]
    n2, e2 = x2.shape[0], adj2_index.shape[1]
    n1a = _rup(n1, 128)
    n2a = _rup(n2, 128)
    n1b = _rup(G * NPG1P, 128)

    dst1p = (adj1_index[0] // NPG1) * NPG1P + (adj1_index[0] % NPG1)
    adj1p = jnp.stack([dst1p, adj1_index[1]])
    zeros = jnp.zeros((max(n1a, n1b, n2a) // 16, H), jnp.float32)

    eye4 = jnp.eye(4, dtype=jnp.float32)
    d1, d2 = x1.shape[1], x2.shape[1]

    s1p, s2p = _tc_call(
        _support_body,
        (jax.ShapeDtypeStruct((n1 // 4, 128), jnp.float32),
         jax.ShapeDtypeStruct((n2 // 4, 128), jnp.float32)),
        x1.reshape(n1 // 4, 4 * d1), jnp.kron(eye4, w1a),
        x2.reshape(n2 // 4, 4 * d2), jnp.kron(eye4, w2a))

    p1, p2 = _make_sc_agg(n1a, n2a, e1, e2)(
        s1p.reshape(n1, H), adj1_index, adj1_weight,
        s2p.reshape(n2, H), adj2_index, adj2_weight, zeros)

    h1p, h2p = _tc_call(
        _mid_body,
        (jax.ShapeDtypeStruct((n1a // 4, 128), jnp.float32),
         jax.ShapeDtypeStruct((n2a // 4, 128), jnp.float32)),
        p1.reshape(2, n1a // 4, 128), jnp.tile(b1a, 4), jnp.kron(eye4, w1b),
        p2.reshape(2, n2a // 4, 128), jnp.tile(b2a, 4), jnp.kron(eye4, w2b))

    q1, q2 = _make_sc_agg(n1b, n2a, e1, e2)(
        h1p.reshape(n1a, H), adj1p, adj1_weight,
        h2p.reshape(n2a, H), adj2_index, adj2_weight, zeros)

    logits = _tc_call(
        _head_body,
        jax.ShapeDtypeStruct((G, 2), jnp.float32),
        q1.reshape(2, n1b // 4, 128), jnp.tile(b1b, 4),
        q2.reshape(2, n2a // 4, 128), jnp.tile(b2b, 4),
        wfc1, bfc1, wfc2, bfc2, wfc3, bfc3)
    return logits


# speedup vs baseline: 18.2280x; 1.1892x over previous
import functools

import jax
import jax.numpy as jnp
from jax import lax
from jax.experimental import pallas as pl
from jax.experimental.pallas import tpu as pltpu
from jax.experimental.pallas import tpu_sc as plsc

H = 32
G = 125
NPG1 = 116
NPG1P = 120
NPG2 = 200
C = 400
NW = 32
_KMAX = 32


def _rup(x, m):
    return (x + m - 1) // m * m


def _support_body(x1, w1, x2, w2, s1, s2):
    s1[...] = jnp.dot(x1[...], w1[...], preferred_element_type=jnp.float32)
    s2[...] = jnp.dot(x2[...], w2[...], preferred_element_type=jnp.float32)


def _mid_body(p1, b1a, w1b, p2, b2a, w2b, h1, h2):
    a1 = jax.nn.relu(p1[0] + p1[1] + b1a[...])
    h1[...] = jnp.dot(a1, w1b[...], preferred_element_type=jnp.float32)
    a2 = jax.nn.relu(p2[0] + p2[1] + b2a[...])
    h2[...] = jnp.dot(a2, w2b[...], preferred_element_type=jnp.float32)


def _fold4(a, op):
    return op(op(a[:, 0:32], a[:, 32:64]), op(a[:, 64:96], a[:, 96:128]))


def _head_body(q1, b1b, q2, b2b, wfc1, bfc1, wfc2, bfc2, wfc3, bfc3, out):
    r1 = G * (NPG1P // 4)
    h1 = jax.nn.relu(q1[0, :r1] + q1[1, :r1] + b1b[...])
    h1 = h1.reshape(G, NPG1P // 4, 128)
    valid = lax.broadcasted_iota(jnp.int32, h1.shape, 1) < NPG1 // 4
    h1m = jnp.where(valid, h1, 0.0)
    mean1 = _fold4(jnp.sum(h1m, axis=1), jnp.add) * (1.0 / NPG1)
    max1 = _fold4(jnp.max(h1m, axis=1), jnp.maximum)

    r2 = G * (NPG2 // 4)
    h2 = jax.nn.relu(q2[0, :r2] + q2[1, :r2] + b2b[...])
    h2 = h2.reshape(G, NPG2 // 4, 128)
    mean2 = _fold4(jnp.sum(h2, axis=1), jnp.add) * (1.0 / NPG2)
    max2 = _fold4(jnp.max(h2, axis=1), jnp.maximum)

    fc = jnp.concatenate([mean1, max1, mean2, max2], axis=1)
    h = jnp.dot(fc, wfc1[...], preferred_element_type=jnp.float32) + bfc1[...]
    h = jnp.dot(h, wfc2[...], preferred_element_type=jnp.float32) + bfc2[...]
    out[...] = jnp.dot(h, wfc3[...], preferred_element_type=jnp.float32) + bfc3[...]


def _tc_call(body, out_shapes, *args):
    return pl.pallas_call(body, out_shape=out_shapes)(*args)


@functools.cache
def _make_sc_agg(n1_acc, n2_acc, e1, e2):
    mesh = plsc.VectorSubcoreMesh(core_axis_name="c", subcore_axis_name="s")
    n_max = max(n1_acc, n2_acc)

    def body(t1, ed1, w1, t2, ed2, w2, zeros, o1, o2,
             idxall, wall, rows0, rows1, acc, si, sg0, sg1, ss0, ss1):
        cid = lax.axis_index("c")
        sid = lax.axis_index("s")
        wid = sid * 2 + cid
        rowsb = (rows0, rows1)
        sg = (sg0, sg1)
        ss = (ss0, ss1)

        def run_branch(tab, edata, wgt, out, n_acc, e):
            rpt = n_acc // 16
            nch = e // C
            k = -(-nch // NW)
            my = (nch + (NW - 1) - wid) >> 5

            for i in range(k):
                @pl.when(i < my)
                def _(i=i):
                    base = (i * NW + wid) * C
                    pltpu.make_async_copy(
                        edata.at[:, pl.ds(base, C)],
                        idxall.at[:, i, :], si).start()
                    pltpu.make_async_copy(
                        wgt.at[pl.ds(base, C)],
                        wall.at[i], si).start()

            pltpu.sync_copy(zeros.at[pl.ds(0, rpt), :],
                            acc.at[pl.ds(sid * rpt, rpt), :])
            for i in range(k):
                @pl.when(i < my)
                def _(i=i):
                    pltpu.make_async_copy(
                        edata.at[:, pl.ds(0, C)],
                        idxall.at[:, i, :], si).wait()
                    pltpu.make_async_copy(
                        wgt.at[pl.ds(0, C)], wall.at[i], si).wait()
            plsc.subcore_barrier()

            def gather(b, i):
                return pltpu.make_async_copy(
                    tab.at[idxall.at[1, i]], rowsb[b], sg[b])

            def scatter(b, i):
                return pltpu.make_async_copy(
                    rowsb[b], acc.at[idxall.at[0, i]], ss[b])

            def scale(b, i):
                rows = rowsb[b]

                def sgroup(g, _):
                    wvec = wall[i, pl.ds(g * 16, 16)]
                    for j in range(16):
                        ei = g * 16 + j
                        w = wvec[j]
                        rows[ei, pl.ds(0, 16)] = rows[ei, pl.ds(0, 16)] * w
                        rows[ei, pl.ds(16, 16)] = rows[ei, pl.ds(16, 16)] * w
                    return 0

                lax.fori_loop(0, C // 16, sgroup, 0)

            def step(i, b):
                nb = 1 - b

                @pl.when(jnp.logical_and(i >= 1, i - 1 < my))
                def _():
                    scatter(nb, i - 1).wait()

                @pl.when(i < my)
                def _():
                    gather(b, i).wait()

                @pl.when(i + 1 < my)
                def _():
                    gather(nb, i + 1).start()

                @pl.when(i < my)
                def _():
                    scale(b, i)
                    scatter(b, i).start(add=True)

            @pl.when(0 < my)
            def _():
                gather(0, 0).start()

            def two_steps(i2, _):
                step(i2 * 2, 0)
                step(i2 * 2 + 1, 1)
                return 0

            lax.fori_loop(0, k // 2, two_steps, 0)
            if k % 2:
                step(k - 1, 0)

            @pl.when(my == k)
            def _():
                scatter((k - 1) % 2, k - 1).wait()

            plsc.subcore_barrier()
            pltpu.sync_copy(acc.at[pl.ds(sid * rpt, rpt), :],
                            out.at[cid, pl.ds(sid * rpt, rpt), :])
            plsc.subcore_barrier()

        run_branch(t1, ed1, w1, o1, n1_acc, e1)
        run_branch(t2, ed2, w2, o2, n2_acc, e2)

    return pl.kernel(
        body,
        out_type=[
            jax.ShapeDtypeStruct((2, n1_acc, H), jnp.float32),
            jax.ShapeDtypeStruct((2, n2_acc, H), jnp.float32),
        ],
        mesh=mesh,
        compiler_params=pltpu.CompilerParams(use_tc_tiling_on_sc=False),
        scratch_types=[
            pltpu.VMEM((2, _KMAX, C), jnp.int32),
            pltpu.VMEM((_KMAX, C), jnp.float32),
            pltpu.VMEM((C, H), jnp.float32),
            pltpu.VMEM((C, H), jnp.float32),
            pltpu.VMEM_SHARED((n_max, H), jnp.float32),
            pltpu.SemaphoreType.DMA,
            pltpu.SemaphoreType.DMA,
            pltpu.SemaphoreType.DMA,
            pltpu.SemaphoreType.DMA,
            pltpu.SemaphoreType.DMA,
        ],
    )


def kernel(x1, adj1_index, adj1_weight, x2, adj2_index, adj2_weight,
           w1a, b1a, w1b, b1b, w2a, b2a, w2b, b2b,
           wfc1, bfc1, wfc2, bfc2, wfc3, bfc3):
    n1, e1 = x1.shape[0], adj1_index.shape[1]
    n2, e2 = x2.shape[0], adj2_index.shape[1]
    n1a = _rup(n1, 128)
    n2a = _rup(n2, 128)
    n1b = _rup(G * NPG1P, 128)

    dst1p = (adj1_index[0] // NPG1) * NPG1P + (adj1_index[0] % NPG1)
    ed1 = adj1_index
    ed1p = jnp.concatenate([dst1p[None], adj1_index[1:2]], axis=0)
    ed2 = adj2_index
    zeros = jnp.zeros((max(n1a, n1b, n2a) // 16, H), jnp.float32)

    eye4 = jnp.eye(4, dtype=jnp.float32)
    d1, d2 = x1.shape[1], x2.shape[1]

    s1p, s2p = _tc_call(
        _support_body,
        (jax.ShapeDtypeStruct((n1 // 4, 128), jnp.float32),
         jax.ShapeDtypeStruct((n2 // 4, 128), jnp.float32)),
        x1.reshape(n1 // 4, 4 * d1), jnp.kron(eye4, w1a),
        x2.reshape(n2 // 4, 4 * d2), jnp.kron(eye4, w2a))

    p1, p2 = _make_sc_agg(n1a, n2a, e1, e2)(
        s1p.reshape(n1, H), ed1, adj1_weight,
        s2p.reshape(n2, H), ed2, adj2_weight, zeros)

    h1p, h2p = _tc_call(
        _mid_body,
        (jax.ShapeDtypeStruct((n1a // 4, 128), jnp.float32),
         jax.ShapeDtypeStruct((n2a // 4, 128), jnp.float32)),
        p1.reshape(2, n1a // 4, 128), jnp.tile(b1a, 4), jnp.kron(eye4, w1b),
        p2.reshape(2, n2a // 4, 128), jnp.tile(b2a, 4), jnp.kron(eye4, w2b))

    q1, q2 = _make_sc_agg(n1b, n2a, e1, e2)(
        h1p.reshape(n1a, H), ed1p, adj1_weight,
        h2p.reshape(n2a, H), ed2, adj2_weight, zeros)

    logits = _tc_call(
        _head_body,
        jax.ShapeDtypeStruct((G, 2), jnp.float32),
        q1.reshape(2, n1b // 4, 128), jnp.tile(b1b, 4),
        q2.reshape(2, n2a // 4, 128), jnp.tile(b2b, 4),
        wfc1, bfc1, wfc2, bfc2, wfc3, bfc3)
    return logits
